# baseline (speedup 1.0000x reference)
import functools

import jax
import jax.numpy as jnp
from jax import lax
from jax.experimental import pallas as pl
from jax.experimental.pallas import tpu as pltpu
from jax.experimental.pallas import tpu_sc as plsc

_NC = 19
_PAD = 320
_BLK = 256
_NB = 64
_CH = 128
_HWB = 16384

def _sqrt16(x):
    magic = jnp.int32(0x5F3759DF)
    xm = jnp.maximum(x, jnp.float32(1e-30))
    i = lax.bitcast_convert_type(xm, jnp.int32)
    y = lax.bitcast_convert_type(magic - lax.shift_right_logical(i, 1),
                                 jnp.float32)
    h = xm * jnp.float32(0.5)
    for _ in range(3):
        y = y * (jnp.float32(1.5) - h * y * y)
    return xm * y


_mesh = plsc.VectorSubcoreMesh(core_axis_name="c", subcore_axis_name="s")


@functools.partial(
    pl.kernel,
    mesh=_mesh,
    compiler_params=pltpu.CompilerParams(needs_layout_passes=False),
    out_type=(
        jax.ShapeDtypeStruct((2, 16, 4096), jnp.float32),
        jax.ShapeDtypeStruct((2, 16, _HWB), jnp.float32),
    ),
    scratch_types=[
        pltpu.VMEM((2, _CH, _BLK), jnp.float32),
        pltpu.VMEM((2, _BLK), jnp.int32),
        pltpu.VMEM((_CH * _PAD,), jnp.float32),
        pltpu.VMEM((_HWB,), jnp.float32),
        pltpu.VMEM((4096,), jnp.float32),
        pltpu.SemaphoreType.DMA,
        pltpu.SemaphoreType.DMA,
        pltpu.SemaphoreType.DMA,
        pltpu.SemaphoreType.DMA,
    ],
)
def _pass1(f_hbm, y_hbm, sums_out, psq_out,
           buf, labbuf, acc, psq, stage, sd0, sd1, sl0, sl1):
    half = lax.axis_index("c")
    b = lax.axis_index("s")
    c0 = half * _CH
    sems_d = (sd0, sd1)
    sems_l = (sl0, sl1)

    zero = jnp.zeros((16,), jnp.float32)

    def _zbody(i, carry):
        acc[pl.ds(i * 16, 16)] = zero
        return carry

    lax.fori_loop(0, (_CH * _PAD) // 16, _zbody, 0)

    def _data_copy(pb, slot):
        return pltpu.make_async_copy(
            f_hbm.at[b, pl.ds(c0, _CH), pl.ds(pb * _BLK, _BLK)],
            buf.at[slot], sems_d[slot])

    def _lab_copy(pb, slot):
        return pltpu.make_async_copy(
            y_hbm.at[b, pl.ds(pb * _BLK, _BLK)],
            labbuf.at[slot], sems_l[slot])

    _data_copy(0, 0).start()
    _lab_copy(0, 0).start()

    iota16 = lax.iota(jnp.int32, 16)
    base = iota16 * jnp.int32(_NC)

    def _outer(g2, carry):
        for s in range(2):
            pb = g2 * 2 + s

            @pl.when(pb + 1 < _NB)
            def _start_next():
                _data_copy(pb + 1, 1 - s).start()
                _lab_copy(pb + 1, 1 - s).start()

            _data_copy(pb, s).wait()
            _lab_copy(pb, s).wait()

            idxs = [base + labbuf[s, pl.ds(j * 16, 16)] for j in range(16)]

            def _cbody(c, ps, s=s, idxs=idxs):
                off = c * jnp.int32(_PAD)
                out = []
                for j in range(16):
                    v = buf[s, c, pl.ds(j * 16, 16)]
                    plsc.addupdate_scatter(acc, [idxs[j] + off], v)
                    out.append(ps[j] + v * v)
                return tuple(out)

            ps = lax.fori_loop(0, _CH, _cbody, (zero,) * 16)
            for j in range(16):
                psq[pl.ds(pb * _BLK + j * 16, 16)] = ps[j]
        return carry

    lax.fori_loop(0, _NB // 2, _outer, 0)

    def _red(c, carry):
        off = c * _PAD
        s0 = zero
        s1 = zero
        for r in range(16):
            s0 = s0 + acc[pl.ds(off + r * _NC, 16)]
            s1 = s1 + acc[pl.ds(off + r * _NC + 16, 16)]
        stage[pl.ds(c * 32, 16)] = s0
        stage[pl.ds(c * 32 + 16, 16)] = s1
        return carry

    lax.fori_loop(0, _CH, _red, 0)

    pltpu.sync_copy(psq, psq_out.at[half, b])
    pltpu.sync_copy(stage, sums_out.at[half, b])


@functools.partial(
    pl.kernel,
    mesh=_mesh,
    compiler_params=pltpu.CompilerParams(needs_layout_passes=False),
    out_type=(
        jax.ShapeDtypeStruct((8192,), jnp.float32),
        jax.ShapeDtypeStruct((1024,), jnp.float32),
        jax.ShapeDtypeStruct((1024,), jnp.float32),
    ),
    scratch_types=[
        pltpu.VMEM((8192,), jnp.float32),
        pltpu.VMEM((8192,), jnp.float32),
        pltpu.VMEM((8192,), jnp.int32),
        pltpu.VMEM((16, 256), jnp.float32),
        pltpu.VMEM((_PAD,), jnp.float32),
        pltpu.VMEM((_PAD,), jnp.float32),
        pltpu.VMEM((256,), jnp.float32),
    ],
)
def _pass2(psq_in, sums_in, y_hbm, fsums_out, np_out, cp_out,
           pbuf0, pbuf1, lbuf, sbuf, nacc, cacc, stage):
    core = lax.axis_index("c")
    sub = lax.axis_index("s")
    t = core * 16 + sub

    zero = jnp.zeros((16,), jnp.float32)
    ones = jnp.ones((16,), jnp.float32)
    iota16 = lax.iota(jnp.int32, 16)
    base = iota16 * jnp.int32(_NC)

    ct0 = t * 8
    lc0 = ct0 - core * _CH
    for bb in range(16):
        pltpu.sync_copy(sums_in.at[core, bb, pl.ds(lc0 * 32, 256)],
                        sbuf.at[bb])
    for cc in range(8):
        s0 = zero
        s1 = zero
        for bb in range(16):
            s0 = s0 + sbuf[bb, pl.ds(cc * 32, 16)]
            s1 = s1 + sbuf[bb, pl.ds(cc * 32 + 16, 16)]
        stage[pl.ds(cc * 32, 16)] = s0
        stage[pl.ds(cc * 32 + 16, 16)] = s1
    pltpu.sync_copy(stage, fsums_out.at[pl.ds(ct0 * 32, 256)])

    off = core * 8192
    pltpu.sync_copy(psq_in.at[0, sub, pl.ds(off, 8192)], pbuf0)
    pltpu.sync_copy(psq_in.at[1, sub, pl.ds(off, 8192)], pbuf1)
    pltpu.sync_copy(y_hbm.at[sub, pl.ds(off, 8192)], lbuf)

    for i in range(_PAD // 16):
        nacc[pl.ds(i * 16, 16)] = zero
        cacc[pl.ds(i * 16, 16)] = zero

    def _nb(i, carry):
        x = pbuf0[pl.ds(i * 16, 16)] + pbuf1[pl.ds(i * 16, 16)]
        r = _sqrt16(x)
        idx = base + lbuf[pl.ds(i * 16, 16)]
        plsc.addupdate_scatter(nacc, [idx], r)
        plsc.addupdate_scatter(cacc, [idx], ones)
        return carry

    lax.fori_loop(0, 8192 // 16, _nb, 0)

    n0 = zero
    n1 = zero
    c0v = zero
    c1v = zero
    for r in range(16):
        n0 = n0 + nacc[pl.ds(r * _NC, 16)]
        n1 = n1 + nacc[pl.ds(r * _NC + 16, 16)]
        c0v = c0v + cacc[pl.ds(r * _NC, 16)]
        c1v = c1v + cacc[pl.ds(r * _NC + 16, 16)]
    stage[pl.ds(0, 16)] = n0
    stage[pl.ds(16, 16)] = n1
    stage[pl.ds(32, 16)] = c0v
    stage[pl.ds(48, 16)] = c1v
    pltpu.sync_copy(stage.at[pl.ds(0, 32)], np_out.at[pl.ds(t * 32, 32)])
    pltpu.sync_copy(stage.at[pl.ds(32, 32)], cp_out.at[pl.ds(t * 32, 32)])


def kernel(feats, y_down):
    B, C, H, W = feats.shape
    f3 = feats.reshape(B, C, H * W)
    y2 = y_down.reshape(B, H * W)

    sums_p, psq_p = _pass1(f3, y2)
    fsums, npart, cpart = _pass2(psq_p, sums_p, y2)

    fs = fsums.reshape(C, 32)[:, :_NC]
    counts = cpart.reshape(32, 32).sum(axis=0)[:_NC]
    nsums = npart.reshape(32, 32).sum(axis=0)[:_NC]
    safe = jnp.maximum(counts, 1.0)
    b_c = (fs / safe[None, :]).T
    n_c = nsums / safe
    return b_c, n_c


# speedup vs baseline: 1.0900x; 1.0900x over previous
import functools

import jax
import jax.numpy as jnp
from jax import lax
from jax.experimental import pallas as pl
from jax.experimental.pallas import tpu as pltpu
from jax.experimental.pallas import tpu_sc as plsc

_NC = 19
_PAD = 320
_BLK = 256
_NB = 64
_CH = 128
_HWB = 16384


def _sqrt16(x):
    magic = jnp.int32(0x5F3759DF)
    xm = jnp.maximum(x, jnp.float32(1e-30))
    i = lax.bitcast_convert_type(xm, jnp.int32)
    y = lax.bitcast_convert_type(magic - lax.shift_right_logical(i, 1),
                                 jnp.float32)
    h = xm * jnp.float32(0.5)
    for _ in range(3):
        y = y * (jnp.float32(1.5) - h * y * y)
    return xm * y


def _lane_transpose_reduce(tmp, iota16):
    zero = jnp.zeros((16,), jnp.float32)
    s0 = zero
    s1 = zero
    for r in range(16):
        g0 = plsc.load_gather(tmp, [iota16 * 16 + r])
        g1 = plsc.load_gather(tmp, [(iota16 + 16) * 16 + r])
        s0 = s0 + g0
        s1 = s1 + g1
    return s0, s1


_mesh = plsc.VectorSubcoreMesh(core_axis_name="c", subcore_axis_name="s")


@functools.partial(
    pl.kernel,
    mesh=_mesh,
    compiler_params=pltpu.CompilerParams(needs_layout_passes=False),
    out_type=(
        jax.ShapeDtypeStruct((2, 16, _CH * _PAD), jnp.float32),
        jax.ShapeDtypeStruct((2, 16, _HWB), jnp.float32),
    ),
    scratch_types=[
        pltpu.VMEM((2, _CH, _BLK), jnp.float32),
        pltpu.VMEM((2, _BLK), jnp.int32),
        pltpu.VMEM((_CH * _PAD,), jnp.float32),
        pltpu.VMEM((_HWB,), jnp.float32),
        pltpu.SemaphoreType.DMA,
        pltpu.SemaphoreType.DMA,
        pltpu.SemaphoreType.DMA,
        pltpu.SemaphoreType.DMA,
    ],
)
def _pass1(f_hbm, y_hbm, acc_out, psq_out,
           buf, labbuf, acc, psq, sd0, sd1, sl0, sl1):
    half = lax.axis_index("c")
    b = lax.axis_index("s")
    c0 = half * _CH
    sems_d = (sd0, sd1)
    sems_l = (sl0, sl1)

    zero = jnp.zeros((16,), jnp.float32)

    def _zbody(i, carry):
        acc[pl.ds(i * 16, 16)] = zero
        return carry

    lax.fori_loop(0, (_CH * _PAD) // 16, _zbody, 0)

    def _data_copy(pb, slot):
        return pltpu.make_async_copy(
            f_hbm.at[b, pl.ds(c0, _CH), pl.ds(pb * _BLK, _BLK)],
            buf.at[slot], sems_d[slot])

    def _lab_copy(pb, slot):
        return pltpu.make_async_copy(
            y_hbm.at[b, pl.ds(pb * _BLK, _BLK)],
            labbuf.at[slot], sems_l[slot])

    _data_copy(0, 0).start()
    _lab_copy(0, 0).start()

    iota16 = lax.iota(jnp.int32, 16)

    def _outer(g2, carry):
        for s in range(2):
            pb = g2 * 2 + s

            @pl.when(pb + 1 < _NB)
            def _start_next():
                _data_copy(pb + 1, 1 - s).start()
                _lab_copy(pb + 1, 1 - s).start()

            _data_copy(pb, s).wait()
            _lab_copy(pb, s).wait()

            idxs = [labbuf[s, pl.ds(j * 16, 16)] * 16 + iota16
                    for j in range(16)]

            def _cbody(c2, ps, s=s, idxs=idxs):
                out = list(ps)
                for u in range(2):
                    c = c2 * 2 + u
                    off = c * jnp.int32(_PAD)
                    for j in range(16):
                        v = buf[s, c, pl.ds(j * 16, 16)]
                        plsc.addupdate_scatter(acc, [idxs[j] + off], v)
                        out[j] = out[j] + v * v
                return tuple(out)

            ps = lax.fori_loop(0, _CH // 2, _cbody, (zero,) * 16)
            for j in range(16):
                psq[pl.ds(pb * _BLK + j * 16, 16)] = ps[j]
        return carry

    lax.fori_loop(0, _NB // 2, _outer, 0)

    pltpu.sync_copy(psq, psq_out.at[half, b])
    pltpu.sync_copy(acc, acc_out.at[half, b])


@functools.partial(
    pl.kernel,
    mesh=_mesh,
    compiler_params=pltpu.CompilerParams(needs_layout_passes=False),
    out_type=(
        jax.ShapeDtypeStruct((8192,), jnp.float32),
        jax.ShapeDtypeStruct((1024,), jnp.float32),
        jax.ShapeDtypeStruct((1024,), jnp.float32),
    ),
    scratch_types=[
        pltpu.VMEM((8192,), jnp.float32),
        pltpu.VMEM((8192,), jnp.float32),
        pltpu.VMEM((8192,), jnp.int32),
        pltpu.VMEM((16, 8 * _PAD), jnp.float32),
        pltpu.VMEM((512,), jnp.float32),
        pltpu.VMEM((512,), jnp.float32),
        pltpu.VMEM((512,), jnp.float32),
        pltpu.VMEM((256,), jnp.float32),
    ],
)
def _pass2(psq_in, acc_in, y_hbm, fsums_out, np_out, cp_out,
           pbuf0, pbuf1, lbuf, abuf, tmp, nacc, cacc, stage):
    core = lax.axis_index("c")
    sub = lax.axis_index("s")
    t = core * 16 + sub

    zero = jnp.zeros((16,), jnp.float32)
    ones = jnp.ones((16,), jnp.float32)
    iota16 = lax.iota(jnp.int32, 16)

    ct0 = t * 8
    lc0 = ct0 - core * _CH
    for bb in range(16):
        pltpu.sync_copy(acc_in.at[core, bb, pl.ds(lc0 * _PAD, 8 * _PAD)],
                        abuf.at[bb])
    for i in range(512 // 16):
        tmp[pl.ds(i * 16, 16)] = zero
    for cc in range(8):
        for k in range(_NC):
            v = zero
            for bb in range(16):
                v = v + abuf[bb, pl.ds(cc * _PAD + k * 16, 16)]
            tmp[pl.ds(k * 16, 16)] = v
        s0, s1 = _lane_transpose_reduce(tmp, iota16)
        stage[pl.ds(cc * 32, 16)] = s0
        stage[pl.ds(cc * 32 + 16, 16)] = s1
    pltpu.sync_copy(stage, fsums_out.at[pl.ds(ct0 * 32, 256)])

    off = core * 8192
    pltpu.sync_copy(psq_in.at[0, sub, pl.ds(off, 8192)], pbuf0)
    pltpu.sync_copy(psq_in.at[1, sub, pl.ds(off, 8192)], pbuf1)
    pltpu.sync_copy(y_hbm.at[sub, pl.ds(off, 8192)], lbuf)

    for i in range(512 // 16):
        nacc[pl.ds(i * 16, 16)] = zero
        cacc[pl.ds(i * 16, 16)] = zero

    def _nb(i, carry):
        x = pbuf0[pl.ds(i * 16, 16)] + pbuf1[pl.ds(i * 16, 16)]
        r = _sqrt16(x)
        idx = lbuf[pl.ds(i * 16, 16)] * 16 + iota16
        plsc.addupdate_scatter(nacc, [idx], r)
        plsc.addupdate_scatter(cacc, [idx], ones)
        return carry

    lax.fori_loop(0, 8192 // 16, _nb, 0)

    n0, n1 = _lane_transpose_reduce(nacc, iota16)
    c0v, c1v = _lane_transpose_reduce(cacc, iota16)
    stage[pl.ds(0, 16)] = n0
    stage[pl.ds(16, 16)] = n1
    stage[pl.ds(32, 16)] = c0v
    stage[pl.ds(48, 16)] = c1v
    pltpu.sync_copy(stage.at[pl.ds(0, 32)], np_out.at[pl.ds(t * 32, 32)])
    pltpu.sync_copy(stage.at[pl.ds(32, 32)], cp_out.at[pl.ds(t * 32, 32)])


def kernel(feats, y_down):
    B, C, H, W = feats.shape
    f3 = feats.reshape(B, C, H * W)
    y2 = y_down.reshape(B, H * W)

    acc_p, psq_p = _pass1(f3, y2)
    fsums, npart, cpart = _pass2(psq_p, acc_p, y2)

    fs = fsums.reshape(C, 32)[:, :_NC]
    counts = cpart.reshape(32, 32).sum(axis=0)[:_NC]
    nsums = npart.reshape(32, 32).sum(axis=0)[:_NC]
    safe = jnp.maximum(counts, 1.0)
    b_c = (fs / safe[None, :]).T
    n_c = nsums / safe
    return b_c, n_c


# speedup vs baseline: 1.3943x; 1.2792x over previous
import functools

import jax
import jax.numpy as jnp
from jax import lax
from jax.experimental import pallas as pl
from jax.experimental.pallas import tpu as pltpu
from jax.experimental.pallas import tpu_sc as plsc

_NC = 19
_PAD = 320
_BLK = 256
_NB = 64
_CH = 128
_HWB = 16384


def _sqrt16(x):
    magic = jnp.int32(0x5F3759DF)
    xm = jnp.maximum(x, jnp.float32(1e-30))
    i = lax.bitcast_convert_type(xm, jnp.int32)
    y = lax.bitcast_convert_type(magic - lax.shift_right_logical(i, 1),
                                 jnp.float32)
    h = xm * jnp.float32(0.5)
    for _ in range(3):
        y = y * (jnp.float32(1.5) - h * y * y)
    return xm * y


def _lane_transpose_reduce(tmp, iota16):
    zero = jnp.zeros((16,), jnp.float32)
    s0 = zero
    s1 = zero
    for r in range(16):
        g0 = plsc.load_gather(tmp, [iota16 * 16 + r])
        g1 = plsc.load_gather(tmp, [(iota16 + 16) * 16 + r])
        s0 = s0 + g0
        s1 = s1 + g1
    return s0, s1


_mesh = plsc.VectorSubcoreMesh(core_axis_name="c", subcore_axis_name="s")


@functools.partial(
    pl.kernel,
    mesh=_mesh,
    compiler_params=pltpu.CompilerParams(needs_layout_passes=False),
    out_type=(
        jax.ShapeDtypeStruct((2, 16, _CH * _PAD), jnp.float32),
        jax.ShapeDtypeStruct((2, 16, _HWB), jnp.float32),
    ),
    scratch_types=[
        pltpu.VMEM((2, _CH, _BLK), jnp.float32),
        pltpu.VMEM((2, _BLK), jnp.int32),
        pltpu.VMEM((_CH * _PAD,), jnp.float32),
        pltpu.VMEM((_HWB,), jnp.float32),
        pltpu.SemaphoreType.DMA,
        pltpu.SemaphoreType.DMA,
        pltpu.SemaphoreType.DMA,
        pltpu.SemaphoreType.DMA,
    ],
)
def _pass1(f_hbm, y_hbm, acc_out, psq_out,
           buf, labbuf, acc, psq, sd0, sd1, sl0, sl1):
    half = lax.axis_index("c")
    b = lax.axis_index("s")
    c0 = half * _CH
    sems_d = (sd0, sd1)
    sems_l = (sl0, sl1)

    zero = jnp.zeros((16,), jnp.float32)

    def _zbody(i, carry):
        acc[pl.ds(i * 16, 16)] = zero
        return carry

    lax.fori_loop(0, (_CH * _PAD) // 16, _zbody, 0)

    def _data_copy(pb, slot):
        return pltpu.make_async_copy(
            f_hbm.at[b, pl.ds(c0, _CH), pl.ds(pb * _BLK, _BLK)],
            buf.at[slot], sems_d[slot])

    def _lab_copy(pb, slot):
        return pltpu.make_async_copy(
            y_hbm.at[b, pl.ds(pb * _BLK, _BLK)],
            labbuf.at[slot], sems_l[slot])

    _data_copy(0, 0).start()
    _lab_copy(0, 0).start()

    iota16 = lax.iota(jnp.int32, 16)

    def _outer(g2, carry):
        for s in range(2):
            pb = g2 * 2 + s

            @pl.when(pb + 1 < _NB)
            def _start_next():
                _data_copy(pb + 1, 1 - s).start()
                _lab_copy(pb + 1, 1 - s).start()

            _data_copy(pb, s).wait()
            _lab_copy(pb, s).wait()

            idxs = [labbuf[s, pl.ds(j * 16, 16)] * 16 + iota16
                    for j in range(16)]

            def _cbody(c, ps, s=s, idxs=idxs):
                off = c * jnp.int32(_PAD)
                out = list(ps)
                for j in range(16):
                    v = buf[s, c, pl.ds(j * 16, 16)]
                    plsc.addupdate_scatter(acc, [idxs[j] + off], v)
                    out[j] = out[j] + v * v
                return tuple(out)

            ps = plsc.parallel_loop(0, _CH, unroll=2,
                                    carry=(zero,) * 16)(_cbody)
            for j in range(16):
                psq[pl.ds(pb * _BLK + j * 16, 16)] = ps[j]
        return carry

    lax.fori_loop(0, _NB // 2, _outer, 0)

    pltpu.sync_copy(psq, psq_out.at[half, b])
    pltpu.sync_copy(acc, acc_out.at[half, b])


@functools.partial(
    pl.kernel,
    mesh=_mesh,
    compiler_params=pltpu.CompilerParams(needs_layout_passes=False),
    out_type=(
        jax.ShapeDtypeStruct((8192,), jnp.float32),
        jax.ShapeDtypeStruct((1024,), jnp.float32),
        jax.ShapeDtypeStruct((1024,), jnp.float32),
    ),
    scratch_types=[
        pltpu.VMEM((8192,), jnp.float32),
        pltpu.VMEM((8192,), jnp.float32),
        pltpu.VMEM((8192,), jnp.int32),
        pltpu.VMEM((16, 8 * _PAD), jnp.float32),
        pltpu.VMEM((512,), jnp.float32),
        pltpu.VMEM((512,), jnp.float32),
        pltpu.VMEM((512,), jnp.float32),
        pltpu.VMEM((256,), jnp.float32),
    ],
)
def _pass2(psq_in, acc_in, y_hbm, fsums_out, np_out, cp_out,
           pbuf0, pbuf1, lbuf, abuf, tmp, nacc, cacc, stage):
    core = lax.axis_index("c")
    sub = lax.axis_index("s")
    t = core * 16 + sub

    zero = jnp.zeros((16,), jnp.float32)
    ones = jnp.ones((16,), jnp.float32)
    iota16 = lax.iota(jnp.int32, 16)

    ct0 = t * 8
    lc0 = ct0 - core * _CH
    for bb in range(16):
        pltpu.sync_copy(acc_in.at[core, bb, pl.ds(lc0 * _PAD, 8 * _PAD)],
                        abuf.at[bb])
    for i in range(512 // 16):
        tmp[pl.ds(i * 16, 16)] = zero
    for cc in range(8):
        for k in range(_NC):
            v = zero
            for bb in range(16):
                v = v + abuf[bb, pl.ds(cc * _PAD + k * 16, 16)]
            tmp[pl.ds(k * 16, 16)] = v
        s0, s1 = _lane_transpose_reduce(tmp, iota16)
        stage[pl.ds(cc * 32, 16)] = s0
        stage[pl.ds(cc * 32 + 16, 16)] = s1
    pltpu.sync_copy(stage, fsums_out.at[pl.ds(ct0 * 32, 256)])

    off = core * 8192
    pltpu.sync_copy(psq_in.at[0, sub, pl.ds(off, 8192)], pbuf0)
    pltpu.sync_copy(psq_in.at[1, sub, pl.ds(off, 8192)], pbuf1)
    pltpu.sync_copy(y_hbm.at[sub, pl.ds(off, 8192)], lbuf)

    for i in range(512 // 16):
        nacc[pl.ds(i * 16, 16)] = zero
        cacc[pl.ds(i * 16, 16)] = zero

    def _nb(i, carry):
        x = pbuf0[pl.ds(i * 16, 16)] + pbuf1[pl.ds(i * 16, 16)]
        r = _sqrt16(x)
        idx = lbuf[pl.ds(i * 16, 16)] * 16 + iota16
        plsc.addupdate_scatter(nacc, [idx], r)
        plsc.addupdate_scatter(cacc, [idx], ones)
        return carry

    lax.fori_loop(0, 8192 // 16, _nb, 0)

    n0, n1 = _lane_transpose_reduce(nacc, iota16)
    c0v, c1v = _lane_transpose_reduce(cacc, iota16)
    stage[pl.ds(0, 16)] = n0
    stage[pl.ds(16, 16)] = n1
    stage[pl.ds(32, 16)] = c0v
    stage[pl.ds(48, 16)] = c1v
    pltpu.sync_copy(stage.at[pl.ds(0, 32)], np_out.at[pl.ds(t * 32, 32)])
    pltpu.sync_copy(stage.at[pl.ds(32, 32)], cp_out.at[pl.ds(t * 32, 32)])


def kernel(feats, y_down):
    B, C, H, W = feats.shape
    f3 = feats.reshape(B, C, H * W)
    y2 = y_down.reshape(B, H * W)

    acc_p, psq_p = _pass1(f3, y2)
    fsums, npart, cpart = _pass2(psq_p, acc_p, y2)

    fs = fsums.reshape(C, 32)[:, :_NC]
    counts = cpart.reshape(32, 32).sum(axis=0)[:_NC]
    nsums = npart.reshape(32, 32).sum(axis=0)[:_NC]
    safe = jnp.maximum(counts, 1.0)
    b_c = (fs / safe[None, :]).T
    n_c = nsums / safe
    return b_c, n_c
